# baseline (speedup 1.0000x reference)
import jax
import jax.numpy as jnp
from jax.experimental import pallas as pl

TARGET_K = 2048


def _copy_body(src_ref, out_ref):
    out_ref[...] = src_ref[...]


def kernel(x, edge_index, target_number_point, W1, b1, W2, b2):
    src = edge_index[0]
    dst = edge_index[1]
    msg = jnp.take(x, src, axis=0)
    agg = jax.ops.segment_sum(msg, dst, num_segments=x.shape[0])
    h = jax.nn.relu(agg @ W1 + b1)
    score = jnp.squeeze(h @ W2 + b2, axis=-1)
    logp = jax.nn.log_softmax(score)
    gumbel = jax.random.gumbel(jax.random.key(42), logp.shape, dtype=logp.dtype)
    zero_k = (jnp.asarray(target_number_point) * 0).astype(logp.dtype)
    _, idx = jax.lax.top_k(logp + gumbel + zero_k, TARGET_K)
    nodes = jnp.take(x, idx, axis=0)
    nodes = pl.pallas_call(
        _copy_body,
        out_shape=jax.ShapeDtypeStruct(nodes.shape, nodes.dtype),
    )(nodes)
    return (score, nodes)


# speedup vs baseline: 1.9293x; 1.9293x over previous
import functools

import jax
import jax.numpy as jnp
from jax.experimental import pallas as pl
from jax.experimental.pallas import tpu as pltpu
from jax.experimental.pallas import tpu_sc as plsc

import numpy as np

TARGET_K = 2048
NC, NS = 2, 16
NW = NC * NS
RANGE = 320
SLOT = RANGE + 8
CHUNK = 8000
CAP = 16384
BATCH = 128


def _perm_tables():
    perm = np.zeros((256,), np.int64)
    cnt = np.zeros((256,), np.int32)
    for b in range(256):
        bits = [i for i in range(8) if b & (1 << i)]
        cnt[b] = len(bits)
        word = 0
        for j, i in enumerate(bits):
            word |= i << (4 * j)
        perm[b] = word
    perm = np.concatenate([perm, np.zeros((16,), np.int64)])
    cnt = np.concatenate([cnt, np.zeros((16,), np.int32)])
    return (jnp.asarray(perm.astype(np.int32)), jnp.asarray(cnt))


_PERMT, _CNTT = _perm_tables()


def _sc_segment_sum(N, D, E):
    n_chunks = E // CHUNK
    mesh = plsc.VectorSubcoreMesh(core_axis_name="c", subcore_axis_name="s")

    @functools.partial(
        pl.kernel, mesh=mesh,
        out_type=jax.ShapeDtypeStruct((NW * RANGE, D), jnp.float32),
        scratch_types=[
            pltpu.VMEM((CHUNK,), jnp.int32),
            pltpu.VMEM((CHUNK,), jnp.int32),
            pltpu.VMEM((CAP,), jnp.int32),
            pltpu.VMEM((CAP,), jnp.int32),
            pltpu.VMEM((CAP // BATCH, BATCH), jnp.int32),
            pltpu.VMEM((272,), jnp.int32),
            pltpu.VMEM((272,), jnp.int32),
            pltpu.VMEM((BATCH, D), jnp.float32),
            pltpu.VMEM_SHARED((NS * SLOT, D), jnp.float32),
            pltpu.SemaphoreType.DMA,
            pltpu.SemaphoreType.DMA,
        ],
    )
    def k(x_hbm, src_hbm, dst_hbm, permt_hbm, cntt_hbm, agg_hbm,
          dstc, srcc, srcstage, dststage, dstbuf, permv, cntv, rows,
          shared, gsem, ssem):
        c = jax.lax.axis_index("c")
        s = jax.lax.axis_index("s")
        wid = c * NS + s
        lo = wid * RANGE
        slot_base = s * SLOT
        trash = slot_base + RANGE

        zero16 = jnp.zeros((16,), jnp.float32)
        def zrow(r, _):
            for l in range(D // 16):
                rows[r, pl.ds(l * 16, 16)] = zero16
            return 0
        jax.lax.fori_loop(0, BATCH, zrow, 0)
        pltpu.sync_copy(rows, shared.at[pl.ds(slot_base, BATCH)])
        pltpu.sync_copy(rows, shared.at[pl.ds(slot_base + BATCH, BATCH)])
        pltpu.sync_copy(rows.at[pl.ds(0, SLOT - 2 * BATCH)],
                        shared.at[pl.ds(slot_base + 2 * BATCH, SLOT - 2 * BATCH)])

        zi16 = jnp.zeros((16,), jnp.int32)
        t16 = jnp.full((16,), 0, jnp.int32) + trash

        def zst(i, _):
            srcstage[pl.ds(i * 16, 16)] = zi16
            dststage[pl.ds(i * 16, 16)] = t16
            return 0
        jax.lax.fori_loop(0, CAP // 16, zst, 0)

        pltpu.sync_copy(permt_hbm, permv)
        pltpu.sync_copy(cntt_hbm, cntv)

        lane = jax.lax.iota(jnp.int32, 16)
        lane4 = lane * 4
        xors = [lane ^ k for k in (1, 2, 4, 8)]
        dn = jax.lax.GatherDimensionNumbers(
            offset_dims=(), collapsed_slice_dims=(0,), start_index_map=(0,))

        def dg(x, idx):
            return jax.lax.gather(
                x, idx[:, None], dn, (1,),
                mode=jax.lax.GatherScatterMode.PROMISE_IN_BOUNDS)

        def chunk_body(ci, pos):
            pltpu.sync_copy(dst_hbm.at[pl.ds(ci * CHUNK, CHUNK)], dstc)
            pltpu.sync_copy(src_hbm.at[pl.ds(ci * CHUNK, CHUNK)], srcc)

            def vec_body(v, pos):
                d = dstc[pl.ds(v * 16, 16)]
                sv = srcc[pl.ds(v * 16, 16)]
                m = (d >= lo) & (d < lo + RANGE)
                w = jnp.where(m, jnp.int32(1), jnp.int32(0)) << lane
                for xv in xors:
                    w = w + dg(w, xv)
                mb = w[0]
                blo = mb & 255
                bhi = (mb >> 8) & 255
                plo = permv[pl.ds(blo, 16)][0]
                phi = permv[pl.ds(bhi, 16)][0]
                clo = cntv[pl.ds(blo, 16)][0]
                chi = cntv[pl.ds(bhi, 16)][0]
                permlo = (plo >> lane4) & 15
                permhi = ((phi >> lane4) & 15) + 8
                g1 = dg(permlo, jnp.minimum(lane, 7))
                g2 = dg(permhi, jnp.clip(lane - clo, 0, 7))
                perm16 = jnp.where(lane < clo, g1, g2)
                dl = d + (slot_base - lo)
                srcstage[pl.ds(pos, 16)] = dg(sv, perm16)
                dststage[pl.ds(pos, 16)] = dg(dl, perm16)
                return jnp.minimum(pos + clo + chi, CAP - 16)

            return jax.lax.fori_loop(0, CHUNK // 16, vec_body, pos)

        cnt = jax.lax.fori_loop(0, n_chunks, chunk_body, jnp.int32(0))
        srcstage[pl.ds(cnt, 16)] = zi16
        dststage[pl.ds(cnt, 16)] = t16
        nb = (cnt + (BATCH - 1)) >> 7

        def cvt(i, _):
            dstbuf[i >> 3, pl.ds((i & 7) * 16, 16)] = dststage[pl.ds(i * 16, 16)]
            return 0
        jax.lax.fori_loop(0, CAP // 16, cvt, 0)

        def batch_body(b, _):
            pltpu.async_copy(x_hbm.at[srcstage.at[pl.ds(b * BATCH, BATCH)]],
                             rows, gsem).wait()
            pltpu.async_copy(rows, shared.at[dstbuf.at[b]], ssem,
                             add=True).wait()
            return 0
        jax.lax.fori_loop(0, nb, batch_body, 0)

        pltpu.sync_copy(shared.at[pl.ds(slot_base, RANGE)],
                        agg_hbm.at[pl.ds(lo, RANGE)])

    return k


def _sc_gather_rows(N, D, B):
    b_per_w = B // NW
    mesh = plsc.VectorSubcoreMesh(core_axis_name="c", subcore_axis_name="s")

    @functools.partial(
        pl.kernel, mesh=mesh,
        out_type=jax.ShapeDtypeStruct((B, D), jnp.float32),
        scratch_types=[
            pltpu.VMEM((b_per_w,), jnp.int32),
            pltpu.VMEM((b_per_w, D), jnp.float32),
            pltpu.SemaphoreType.DMA,
        ],
    )
    def k(x_hbm, idx_hbm, out_hbm, idx_v, rows_v, sem):
        wid = jax.lax.axis_index("s") * NC + jax.lax.axis_index("c")
        base = wid * b_per_w
        pltpu.sync_copy(idx_hbm.at[pl.ds(base, b_per_w)], idx_v)
        pltpu.async_copy(x_hbm.at[idx_v], rows_v, sem).wait()
        pltpu.sync_copy(rows_v, out_hbm.at[pl.ds(base, b_per_w)])

    return k


def kernel(x, edge_index, target_number_point, W1, b1, W2, b2):
    N, D = x.shape
    src = edge_index[0].astype(jnp.int32)
    dst = edge_index[1].astype(jnp.int32)
    E = src.shape[0]

    aggp = _sc_segment_sum(N, D, E)(x, src, dst, _PERMT, _CNTT)
    agg = aggp[:N]

    h = jax.nn.relu(agg @ W1 + b1)
    score = jnp.squeeze(h @ W2 + b2, axis=-1)
    logp = jax.nn.log_softmax(score)
    gumbel = jax.random.gumbel(jax.random.key(42), logp.shape, dtype=logp.dtype)
    zero_k = (jnp.asarray(target_number_point) * 0).astype(logp.dtype)
    _, idx = jax.lax.top_k(logp + gumbel + zero_k, TARGET_K)
    nodes = _sc_gather_rows(N, D, TARGET_K)(x, idx.astype(jnp.int32))
    return (score, nodes)


# speedup vs baseline: 2.1698x; 1.1247x over previous
import functools

import jax
import jax.numpy as jnp
from jax.experimental import pallas as pl
from jax.experimental.pallas import tpu as pltpu
from jax.experimental.pallas import tpu_sc as plsc

import numpy as np

TARGET_K = 2048
NC, NS = 2, 16
NW = NC * NS
RANGE = 320
SLOT = RANGE + 8
CHUNK = 2000
CAP = 12288
BATCH = 128


def _perm_tables():
    perm = np.zeros((256,), np.int64)
    cnt = np.zeros((256,), np.int32)
    for b in range(256):
        bits = [i for i in range(8) if b & (1 << i)]
        cnt[b] = len(bits)
        word = 0
        for j, i in enumerate(bits):
            word |= i << (4 * j)
        perm[b] = word
    perm = np.concatenate([perm, np.zeros((16,), np.int64)])
    cnt = np.concatenate([cnt, np.zeros((16,), np.int32)])
    return (jnp.asarray(perm.astype(np.int32)), jnp.asarray(cnt))


_PERMT, _CNTT = _perm_tables()


def _sc_segment_sum(N, D, E):
    n_chunks = E // CHUNK
    mesh = plsc.VectorSubcoreMesh(core_axis_name="c", subcore_axis_name="s")

    @functools.partial(
        pl.kernel, mesh=mesh,
        out_type=jax.ShapeDtypeStruct((NW * RANGE, D), jnp.float32),
        scratch_types=[
            pltpu.VMEM((CHUNK,), jnp.int32),
            pltpu.VMEM((CHUNK,), jnp.int32),
            pltpu.VMEM((CHUNK,), jnp.int32),
            pltpu.VMEM((CHUNK,), jnp.int32),
            pltpu.VMEM((CAP,), jnp.int32),
            pltpu.VMEM((CAP,), jnp.int32),
            pltpu.VMEM((CAP // BATCH, BATCH), jnp.int32),
            pltpu.VMEM((272,), jnp.int32),
            pltpu.VMEM((272,), jnp.int32),
            pltpu.VMEM((BATCH, D), jnp.float32),
            pltpu.VMEM((BATCH, D), jnp.float32),
            pltpu.VMEM_SHARED((NS * SLOT, D), jnp.float32),
            pltpu.SemaphoreType.DMA,
            pltpu.SemaphoreType.DMA,
            pltpu.SemaphoreType.DMA,
            pltpu.SemaphoreType.DMA,
            pltpu.SemaphoreType.DMA,
        ],
    )
    def k(x_hbm, src_hbm, dst_hbm, permt_hbm, cntt_hbm, agg_hbm,
          dstc, srcc, dstc1, srcc1, srcstage, dststage, dstbuf, permv, cntv,
          rows, rows1, shared, gsem, gsem1, ssem, csem, csem1):
        c = jax.lax.axis_index("c")
        s = jax.lax.axis_index("s")
        wid = c * NS + s
        lo = wid * RANGE
        slot_base = s * SLOT
        trash = slot_base + RANGE

        zero16 = jnp.zeros((16,), jnp.float32)
        def zrow(r, _):
            for l in range(D // 16):
                rows[r, pl.ds(l * 16, 16)] = zero16
            return 0
        jax.lax.fori_loop(0, BATCH, zrow, 0)
        pltpu.sync_copy(rows, shared.at[pl.ds(slot_base, BATCH)])
        pltpu.sync_copy(rows, shared.at[pl.ds(slot_base + BATCH, BATCH)])
        pltpu.sync_copy(rows.at[pl.ds(0, SLOT - 2 * BATCH)],
                        shared.at[pl.ds(slot_base + 2 * BATCH, SLOT - 2 * BATCH)])

        zi16 = jnp.zeros((16,), jnp.int32)
        t16 = jnp.full((16,), 0, jnp.int32) + trash

        def zst(i, _):
            srcstage[pl.ds(i * 16, 16)] = zi16
            dststage[pl.ds(i * 16, 16)] = t16
            return 0
        jax.lax.fori_loop(0, CAP // 16, zst, 0)

        pltpu.sync_copy(permt_hbm, permv)
        pltpu.sync_copy(cntt_hbm, cntv)

        lane = jax.lax.iota(jnp.int32, 16)
        lane4 = lane * 4
        xors = [lane ^ k for k in (1, 2, 4, 8)]
        dn = jax.lax.GatherDimensionNumbers(
            offset_dims=(), collapsed_slice_dims=(0,), start_index_map=(0,))

        def dg(x, idx):
            return jax.lax.gather(
                x, idx[:, None], dn, (1,),
                mode=jax.lax.GatherScatterMode.PROMISE_IN_BOUNDS)

        def start_c(ci, dbuf, sbuf, sem):
            pltpu.async_copy(dst_hbm.at[pl.ds(ci * CHUNK, CHUNK)], dbuf, sem)
            pltpu.async_copy(src_hbm.at[pl.ds(ci * CHUNK, CHUNK)], sbuf, sem)

        def wait_c(dbuf, sbuf, sem):
            pltpu.make_async_copy(dst_hbm.at[pl.ds(0, CHUNK)], dbuf, sem).wait()
            pltpu.make_async_copy(src_hbm.at[pl.ds(0, CHUNK)], sbuf, sem).wait()

        def chunk_body(ci, pos, dbuf, sbuf):
            def vec_body(v, pos):
                d = dbuf[pl.ds(v * 16, 16)]
                sv = sbuf[pl.ds(v * 16, 16)]
                m = (d >= lo) & (d < lo + RANGE)
                w = jnp.where(m, jnp.int32(1), jnp.int32(0)) << lane
                for xv in xors:
                    w = w + dg(w, xv)
                mb = w[0]
                blo = mb & 255
                bhi = (mb >> 8) & 255
                plo = permv[pl.ds(blo, 16)][0]
                phi = permv[pl.ds(bhi, 16)][0]
                clo = cntv[pl.ds(blo, 16)][0]
                chi = cntv[pl.ds(bhi, 16)][0]
                permlo = (plo >> lane4) & 15
                permhi = ((phi >> lane4) & 15) + 8
                g1 = dg(permlo, jnp.minimum(lane, 7))
                g2 = dg(permhi, jnp.clip(lane - clo, 0, 7))
                perm16 = jnp.where(lane < clo, g1, g2)
                dl = d + (slot_base - lo)
                srcstage[pl.ds(pos, 16)] = dg(sv, perm16)
                dststage[pl.ds(pos, 16)] = dg(dl, perm16)
                return jnp.minimum(pos + clo + chi, CAP - 16)

            return jax.lax.fori_loop(0, CHUNK // 16, vec_body, pos)

        start_c(0, dstc, srcc, csem)

        def chunk_pair(i, pos):
            c0 = 2 * i
            wait_c(dstc, srcc, csem)

            @pl.when(c0 + 1 < n_chunks)
            def _():
                start_c(c0 + 1, dstc1, srcc1, csem1)
            pos = chunk_body(c0, pos, dstc, srcc)

            @pl.when(c0 + 2 < n_chunks)
            def _():
                start_c(c0 + 2, dstc, srcc, csem)
            wait_c(dstc1, srcc1, csem1)
            pos = chunk_body(c0 + 1, pos, dstc1, srcc1)
            return pos

        cnt = jax.lax.fori_loop(0, n_chunks // 2, chunk_pair, jnp.int32(0))
        srcstage[pl.ds(cnt, 16)] = zi16
        dststage[pl.ds(cnt, 16)] = t16
        nb = (cnt + (BATCH - 1)) >> 7

        def cvt(i, _):
            dstbuf[i >> 3, pl.ds((i & 7) * 16, 16)] = dststage[pl.ds(i * 16, 16)]
            return 0
        jax.lax.fori_loop(0, CAP // 16, cvt, 0)

        nb = jnp.maximum(nb, 1)

        def start_g(b, buf, sem):
            pltpu.async_copy(x_hbm.at[srcstage.at[pl.ds(b * BATCH, BATCH)]],
                             buf, sem)

        def wait_g(buf, sem):
            pltpu.make_async_copy(x_hbm.at[pl.ds(0, BATCH)], buf, sem).wait()

        def scat(b, buf):
            pltpu.async_copy(buf, shared.at[dstbuf.at[b]], ssem,
                             add=True).wait()

        start_g(0, rows, gsem)
        nb2 = (nb + 1) >> 1

        def batch_body(i, _):
            b0 = 2 * i

            @pl.when(b0 < nb)
            def _():
                wait_g(rows, gsem)

                @pl.when(b0 + 1 < nb)
                def _():
                    start_g(b0 + 1, rows1, gsem1)
                scat(b0, rows)

            @pl.when(b0 + 1 < nb)
            def _():
                wait_g(rows1, gsem1)

                @pl.when(b0 + 2 < nb)
                def _():
                    start_g(b0 + 2, rows, gsem)
                scat(b0 + 1, rows1)
            return 0
        jax.lax.fori_loop(0, nb2, batch_body, 0)

        pltpu.sync_copy(shared.at[pl.ds(slot_base, RANGE)],
                        agg_hbm.at[pl.ds(lo, RANGE)])

    return k


def _sc_gather_rows(N, D, B):
    b_per_w = B // NW
    mesh = plsc.VectorSubcoreMesh(core_axis_name="c", subcore_axis_name="s")

    @functools.partial(
        pl.kernel, mesh=mesh,
        out_type=jax.ShapeDtypeStruct((B, D), jnp.float32),
        scratch_types=[
            pltpu.VMEM((b_per_w,), jnp.int32),
            pltpu.VMEM((b_per_w, D), jnp.float32),
            pltpu.SemaphoreType.DMA,
        ],
    )
    def k(x_hbm, idx_hbm, out_hbm, idx_v, rows_v, sem):
        wid = jax.lax.axis_index("s") * NC + jax.lax.axis_index("c")
        base = wid * b_per_w
        pltpu.sync_copy(idx_hbm.at[pl.ds(base, b_per_w)], idx_v)
        pltpu.async_copy(x_hbm.at[idx_v], rows_v, sem).wait()
        pltpu.sync_copy(rows_v, out_hbm.at[pl.ds(base, b_per_w)])

    return k


def _tc_score(Np, D):
    def body(agg_ref, W1_ref, b1_ref, W2_ref, b2_ref, out_ref):
        h = jax.nn.relu(
            jnp.dot(agg_ref[...], W1_ref[...],
                    preferred_element_type=jnp.float32) + b1_ref[...])
        s = jnp.dot(h, W2_ref[...],
                    preferred_element_type=jnp.float32) + b2_ref[...]
        out_ref[...] = s[:, 0]

    return pl.pallas_call(
        body, out_shape=jax.ShapeDtypeStruct((Np,), jnp.float32))


def kernel(x, edge_index, target_number_point, W1, b1, W2, b2):
    N, D = x.shape
    src = edge_index[0].astype(jnp.int32)
    dst = edge_index[1].astype(jnp.int32)
    E = src.shape[0]

    aggp = _sc_segment_sum(N, D, E)(x, src, dst, _PERMT, _CNTT)

    score = _tc_score(aggp.shape[0], D)(aggp, W1, b1, W2, b2)[:N]
    logp = jax.nn.log_softmax(score)
    gumbel = jax.random.gumbel(jax.random.key(42), logp.shape, dtype=logp.dtype)
    zero_k = (jnp.asarray(target_number_point) * 0).astype(logp.dtype)
    _, idx = jax.lax.top_k(logp + gumbel + zero_k, TARGET_K)
    nodes = _sc_gather_rows(N, D, TARGET_K)(x, idx.astype(jnp.int32))
    return (score, nodes)
